# baseline (speedup 1.0000x reference)
import functools

import jax
import jax.numpy as jnp
from jax import lax
from jax.experimental import pallas as pl
from jax.experimental.pallas import tpu as pltpu
from jax.experimental.pallas import tpu_sc as plsc

N_DATA = 16384
N_SUP = 32768
DB = 512
SB = 2048

SC_CORES = 2
SC_SUBCORES = 16
ROWS_PER_TILE = 8
LANES = 128


def _argmin_body(saug_ref, daugt_ref, iota_ref, out_ref, rmin_s, ridx_s):
    j = pl.program_id(1)

    @pl.when(j == 0)
    def _init():
        rmin_s[...] = jnp.full((SB,), jnp.inf, jnp.float32)
        ridx_s[...] = jnp.zeros((SB,), jnp.int32)

    t = jnp.dot(saug_ref[...], daugt_ref[...],
                preferred_element_type=jnp.float32,
                precision=jax.lax.Precision.HIGHEST)
    m = jnp.min(t, axis=1)
    cand = jnp.min(jnp.where(t == m[:, None], iota_ref[...], jnp.float32(1e9)),
                   axis=1)
    candi = cand.astype(jnp.int32)
    prev = rmin_s[...]
    upd = m < prev
    rmin_s[...] = jnp.where(upd, m, prev)
    ridx_s[...] = jnp.where(upd, candi, ridx_s[...])

    @pl.when(j == pl.num_programs(1) - 1)
    def _emit():
        out_ref[...] = ridx_s[...]


def _tc_nearest(saug, daugt, iota_f):
    return pl.pallas_call(
        _argmin_body,
        grid=(N_SUP // SB, N_DATA // DB),
        in_specs=[
            pl.BlockSpec((SB, 8), lambda i, j: (i, 0)),
            pl.BlockSpec((8, DB), lambda i, j: (0, j)),
            pl.BlockSpec((1, DB), lambda i, j: (0, j)),
        ],
        out_specs=pl.BlockSpec((SB,), lambda i, j: (i,)),
        out_shape=jax.ShapeDtypeStruct((N_SUP,), jnp.int32),
        scratch_shapes=[
            pltpu.VMEM((SB,), jnp.float32),
            pltpu.VMEM((SB,), jnp.int32),
        ],
        compiler_params=pltpu.CompilerParams(
            dimension_semantics=("parallel", "arbitrary")),
    )(saug, daugt, iota_f)


def _sc_hist(nearest3, zeros_init):
    mesh = plsc.VectorSubcoreMesh(core_axis_name="c", subcore_axis_name="s")

    @functools.partial(
        pl.kernel,
        mesh=mesh,
        out_type=jax.ShapeDtypeStruct((SC_CORES, N_DATA), jnp.float32),
        scratch_types=[
            pltpu.VMEM((ROWS_PER_TILE, LANES), jnp.int32),
            pltpu.VMEM((LANES,), jnp.float32),
            pltpu.VMEM_SHARED((N_DATA,), jnp.float32),
        ],
    )
    def hist(near_hbm, z_hbm, out_hbm, idx_v, val_v, shared):
        c = lax.axis_index("c")
        s = lax.axis_index("s")
        wid = c * SC_SUBCORES + s

        @pl.when(s == 0)
        def _zero():
            pltpu.sync_copy(z_hbm, shared)

        for i in range(LANES // 16):
            val_v[pl.ds(i * 16, 16)] = jnp.full((16,), 1.0, jnp.float32)
        pltpu.sync_copy(near_hbm.at[wid], idx_v)
        plsc.subcore_barrier()
        for r in range(ROWS_PER_TILE):
            pltpu.sync_copy(val_v, shared.at[idx_v.at[r]], add=True)
        plsc.subcore_barrier()

        @pl.when(s == 0)
        def _emit():
            pltpu.sync_copy(shared, out_hbm.at[c])

    return hist(nearest3, zeros_init)


def kernel(data_points, grid):
    dp = data_points.astype(jnp.float32)
    g = grid.astype(jnp.float32)
    dsq = jnp.sum(dp * dp, axis=1, keepdims=True)
    saug = jnp.concatenate(
        [-2.0 * g, jnp.ones((N_SUP, 1), jnp.float32),
         jnp.zeros((N_SUP, 4), jnp.float32)], axis=1)
    daugt = jnp.concatenate(
        [dp.T, dsq.T, jnp.zeros((4, N_DATA), jnp.float32)], axis=0)
    iota_f = jnp.arange(N_DATA, dtype=jnp.float32).reshape(1, N_DATA)
    nearest = _tc_nearest(saug, daugt, iota_f)
    h = _sc_hist(
        nearest.reshape(SC_CORES * SC_SUBCORES, ROWS_PER_TILE, LANES),
        jnp.zeros((N_DATA,), jnp.float32))
    return (h[0] + h[1]) * jnp.float32(1.0 / N_SUP)


# speedup vs baseline: 2.1296x; 2.1296x over previous
import functools

import jax
import jax.numpy as jnp
from jax import lax
from jax.experimental import pallas as pl
from jax.experimental.pallas import tpu as pltpu
from jax.experimental.pallas import tpu_sc as plsc

N_DATA = 16384
N_SUP = 32768
DB = 2048
CH = 128
SB = 1024

SC_CORES = 2
SC_SUBCORES = 16
ROWS_PER_TILE = 8
LANES = 128


def _argmin_body(daug_ref, saugt_ref, out_ref, rmin_s, ridx_s):
    j = pl.program_id(1)

    @pl.when(j == 0)
    def _init():
        rmin_s[...] = jnp.full((SB,), jnp.inf, jnp.float32)
        ridx_s[...] = jnp.zeros((SB,), jnp.int32)

    saugt = saugt_ref[...]
    rowsf = lax.broadcasted_iota(jnp.int32, (CH, SB), 0).astype(jnp.float32)
    ms, cands = [], []
    for c in range(DB // CH):
        t = jnp.dot(daug_ref[pl.ds(c * CH, CH), :], saugt,
                    preferred_element_type=jnp.float32,
                    precision=jax.lax.Precision.HIGHEST)
        mc = jnp.min(t, axis=0)
        cc = jnp.min(jnp.where(t == mc[None, :], rowsf, jnp.float32(1e9)),
                     axis=0) + jnp.float32(c * CH)
        ms.append(mc)
        cands.append(cc)
    m = ms[0]
    cand = cands[0]
    for mc, cc in zip(ms[1:], cands[1:]):
        cand = jnp.where(mc < m, cc, cand)
        m = jnp.minimum(m, mc)
    candi = cand.astype(jnp.int32) + j * DB
    prev = rmin_s[...]
    upd = m < prev
    rmin_s[...] = jnp.where(upd, m, prev)
    ridx_s[...] = jnp.where(upd, candi, ridx_s[...])

    @pl.when(j == pl.num_programs(1) - 1)
    def _emit():
        out_ref[...] = ridx_s[...]


def _tc_nearest(daug, saugt):
    return pl.pallas_call(
        _argmin_body,
        grid=(N_SUP // SB, N_DATA // DB),
        in_specs=[
            pl.BlockSpec((DB, 8), lambda i, j: (j, 0)),
            pl.BlockSpec((8, SB), lambda i, j: (0, i)),
        ],
        out_specs=pl.BlockSpec((SB,), lambda i, j: (i,)),
        out_shape=jax.ShapeDtypeStruct((N_SUP,), jnp.int32),
        scratch_shapes=[
            pltpu.VMEM((SB,), jnp.float32),
            pltpu.VMEM((SB,), jnp.int32),
        ],
        compiler_params=pltpu.CompilerParams(
            dimension_semantics=("parallel", "arbitrary")),
    )(daug, saugt)


def _sc_hist(nearest3, zeros_init):
    mesh = plsc.VectorSubcoreMesh(core_axis_name="c", subcore_axis_name="s")

    @functools.partial(
        pl.kernel,
        mesh=mesh,
        out_type=jax.ShapeDtypeStruct((SC_CORES, N_DATA), jnp.float32),
        scratch_types=[
            pltpu.VMEM((ROWS_PER_TILE, LANES), jnp.int32),
            pltpu.VMEM((LANES,), jnp.float32),
            pltpu.VMEM_SHARED((N_DATA,), jnp.float32),
        ],
    )
    def hist(near_hbm, z_hbm, out_hbm, idx_v, val_v, shared):
        c = lax.axis_index("c")
        s = lax.axis_index("s")
        wid = c * SC_SUBCORES + s

        @pl.when(s == 0)
        def _zero():
            pltpu.sync_copy(z_hbm, shared)

        for i in range(LANES // 16):
            val_v[pl.ds(i * 16, 16)] = jnp.full((16,), 1.0, jnp.float32)
        pltpu.sync_copy(near_hbm.at[wid], idx_v)
        plsc.subcore_barrier()
        for r in range(ROWS_PER_TILE):
            pltpu.sync_copy(val_v, shared.at[idx_v.at[r]], add=True)
        plsc.subcore_barrier()

        @pl.when(s == 0)
        def _emit():
            pltpu.sync_copy(shared, out_hbm.at[c])

    return hist(nearest3, zeros_init)


def kernel(data_points, grid):
    dp = data_points.astype(jnp.float32)
    g = grid.astype(jnp.float32)
    dsq = jnp.sum(dp * dp, axis=1, keepdims=True)
    daug = jnp.concatenate(
        [dp, dsq, jnp.zeros((N_DATA, 4), jnp.float32)], axis=1)
    saugt = jnp.concatenate(
        [(-2.0 * g).T,
         jnp.ones((1, N_SUP), jnp.float32),
         jnp.zeros((4, N_SUP), jnp.float32)], axis=0)
    nearest = _tc_nearest(daug, saugt)
    h = _sc_hist(
        nearest.reshape(SC_CORES * SC_SUBCORES, ROWS_PER_TILE, LANES),
        jnp.zeros((N_DATA,), jnp.float32))
    return (h[0] + h[1]) * jnp.float32(1.0 / N_SUP)


# speedup vs baseline: 2.1780x; 1.0227x over previous
import functools

import jax
import jax.numpy as jnp
from jax import lax
from jax.experimental import pallas as pl
from jax.experimental.pallas import tpu as pltpu
from jax.experimental.pallas import tpu_sc as plsc

N_DATA = 16384
N_SUP = 32768
DB = 2048
CH = 128
SB = 2048

SC_CORES = 2
SC_SUBCORES = 16
ROWS_PER_TILE = 8
LANES = 128


def _argmin_body(daug_ref, saugt_ref, out_ref, rmin_s, ridx_s):
    j = pl.program_id(1)

    @pl.when(j == 0)
    def _init():
        rmin_s[...] = jnp.full((SB,), jnp.inf, jnp.float32)
        ridx_s[...] = jnp.zeros((SB,), jnp.int32)

    saugt = saugt_ref[...]
    rowsf = lax.broadcasted_iota(jnp.int32, (CH, SB), 0).astype(jnp.float32)
    ms, cands = [], []
    for c in range(DB // CH):
        t = jnp.dot(daug_ref[pl.ds(c * CH, CH), :], saugt,
                    preferred_element_type=jnp.float32,
                    precision=jax.lax.Precision.HIGHEST)
        mc = jnp.min(t, axis=0)
        cc = jnp.min(jnp.where(t == mc[None, :], rowsf, jnp.float32(1e9)),
                     axis=0) + jnp.float32(c * CH)
        ms.append(mc)
        cands.append(cc)
    m = ms[0]
    cand = cands[0]
    for mc, cc in zip(ms[1:], cands[1:]):
        cand = jnp.where(mc < m, cc, cand)
        m = jnp.minimum(m, mc)
    candi = cand.astype(jnp.int32) + j * DB
    prev = rmin_s[...]
    upd = m < prev
    rmin_s[...] = jnp.where(upd, m, prev)
    ridx_s[...] = jnp.where(upd, candi, ridx_s[...])

    @pl.when(j == pl.num_programs(1) - 1)
    def _emit():
        out_ref[...] = ridx_s[...]


def _tc_nearest(daug, saugt):
    return pl.pallas_call(
        _argmin_body,
        grid=(N_SUP // SB, N_DATA // DB),
        in_specs=[
            pl.BlockSpec((DB, 8), lambda i, j: (j, 0)),
            pl.BlockSpec((8, SB), lambda i, j: (0, i)),
        ],
        out_specs=pl.BlockSpec((SB,), lambda i, j: (i,)),
        out_shape=jax.ShapeDtypeStruct((N_SUP,), jnp.int32),
        scratch_shapes=[
            pltpu.VMEM((SB,), jnp.float32),
            pltpu.VMEM((SB,), jnp.int32),
        ],
        compiler_params=pltpu.CompilerParams(
            dimension_semantics=("parallel", "arbitrary")),
    )(daug, saugt)


def _sc_hist(nearest3, zeros_init):
    mesh = plsc.VectorSubcoreMesh(core_axis_name="c", subcore_axis_name="s")

    @functools.partial(
        pl.kernel,
        mesh=mesh,
        out_type=jax.ShapeDtypeStruct((SC_CORES, N_DATA), jnp.float32),
        scratch_types=[
            pltpu.VMEM((ROWS_PER_TILE, LANES), jnp.int32),
            pltpu.VMEM((LANES,), jnp.float32),
            pltpu.VMEM_SHARED((N_DATA,), jnp.float32),
        ],
    )
    def hist(near_hbm, z_hbm, out_hbm, idx_v, val_v, shared):
        c = lax.axis_index("c")
        s = lax.axis_index("s")
        wid = c * SC_SUBCORES + s

        @pl.when(s == 0)
        def _zero():
            pltpu.sync_copy(z_hbm, shared)

        for i in range(LANES // 16):
            val_v[pl.ds(i * 16, 16)] = jnp.full((16,), 1.0, jnp.float32)
        pltpu.sync_copy(near_hbm.at[wid], idx_v)
        plsc.subcore_barrier()
        for r in range(ROWS_PER_TILE):
            pltpu.sync_copy(val_v, shared.at[idx_v.at[r]], add=True)
        plsc.subcore_barrier()

        @pl.when(s == 0)
        def _emit():
            pltpu.sync_copy(shared, out_hbm.at[c])

    return hist(nearest3, zeros_init)


def kernel(data_points, grid):
    dp = data_points.astype(jnp.float32)
    g = grid.astype(jnp.float32)
    dsq = jnp.sum(dp * dp, axis=1, keepdims=True)
    daug = jnp.concatenate(
        [dp, dsq, jnp.zeros((N_DATA, 4), jnp.float32)], axis=1)
    saugt = jnp.concatenate(
        [(-2.0 * g).T,
         jnp.ones((1, N_SUP), jnp.float32),
         jnp.zeros((4, N_SUP), jnp.float32)], axis=0)
    nearest = _tc_nearest(daug, saugt)
    h = _sc_hist(
        nearest.reshape(SC_CORES * SC_SUBCORES, ROWS_PER_TILE, LANES),
        jnp.zeros((N_DATA,), jnp.float32))
    return (h[0] + h[1]) * jnp.float32(1.0 / N_SUP)
